# baseline (speedup 1.0000x reference)
import functools

import jax
import jax.numpy as jnp
from jax import lax
from jax.experimental import pallas as pl
from jax.experimental.pallas import tpu as pltpu
from jax.experimental.pallas import tpu_sc as plsc

NPAD = 10240
EPAD = 163840
CW = 64
ER = EPAD // CW
NC, NS = 2, 16
RPC = ER // NS
STG = 40
RN = NPAD // NS
RADIX = 16384.0
LCLAMP = -30.0

_MESH = plsc.VectorSubcoreMesh(
    core_axis_name="c", subcore_axis_name="s", num_cores=NC, num_subcores=NS)


def _sc_degrees(src2, dst2):
  @functools.partial(
      pl.kernel,
      out_type=[jax.ShapeDtypeStruct((NPAD, 128), jnp.float32),
                jax.ShapeDtypeStruct((NPAD, 128), jnp.float32)],
      mesh=_MESH,
      scratch_types=[
          pltpu.VMEM_SHARED((NPAD, 128), jnp.float32),
          pltpu.VMEM((RPC, CW), jnp.int32),
          pltpu.VMEM((CW, 128), jnp.float32),
          pltpu.VMEM((CW, 128), jnp.float32),
          pltpu.SemaphoreType.DMA,
          pltpu.SemaphoreType.DMA,
      ],
  )
  def deg_kernel(src_hbm, dst_hbm, out0_hbm, out1_hbm, acc, idx, ones, zbuf,
                 sa, sb):
    c = lax.axis_index("c")
    s = lax.axis_index("s")

    def fill(i, _):
      for f in range(8):
        d = pl.ds(f * 16, 16)
        ones[i, d] = jnp.full((16,), 1.0, jnp.float32)
        zbuf[i, d] = jnp.zeros((16,), jnp.float32)
      return 0
    lax.fori_loop(0, CW, fill, 0)

    def zacc(t, _):
      pltpu.sync_copy(zbuf, acc.at[pl.ds(s * RN + t * CW, CW)])
      return 0
    lax.fori_loop(0, RN // CW, zacc, 0)
    plsc.subcore_barrier()

    def scan(idx_hbm):
      pltpu.sync_copy(idx_hbm.at[pl.ds(s * RPC, RPC)], idx)
      sems = [sa, sb]
      descs = [None, None]
      for j in range(RPC):
        pj = j % 2
        if descs[pj] is not None:
          descs[pj].wait()
        descs[pj] = pltpu.async_copy(ones, acc.at[idx.at[j]], sems[pj],
                                     add=True)
      for d_ in descs:
        if d_ is not None:
          d_.wait()

    @pl.when(c == 0)
    def _():
      scan(src_hbm)

    @pl.when(c == 1)
    def _():
      scan(dst_hbm)

    plsc.subcore_barrier()

    @pl.when(c == 0)
    def _():
      pltpu.sync_copy(acc.at[pl.ds(s * RN, RN)], out0_hbm.at[pl.ds(s * RN, RN)])

    @pl.when(c == 1)
    def _():
      pltpu.sync_copy(acc.at[pl.ds(s * RN, RN)], out1_hbm.at[pl.ds(s * RN, RN)])

  return deg_kernel(src2, dst2)


def _tc_node(featp, w1, w2a, b2, outdeg):
  blk = 512
  grid = NPAD // blk

  def body(x_ref, w1_ref, w2a_ref, b2_ref, od_ref, a1_ref, l_ref):
    x = x_ref[...]
    s = lax.rsqrt(jnp.clip(od_ref[...][:, 0:1], 1.0, None))
    a1 = jnp.dot(x, w1_ref[...], preferred_element_type=jnp.float32) * s
    z = jnp.dot(x, w2a_ref[...], preferred_element_type=jnp.float32) * s
    p = jnp.tanh(z + b2_ref[...])
    a1_ref[...] = a1
    negp = jnp.where(p < 0, 1.0, 0.0).astype(jnp.float32)
    l_ref[...] = jnp.maximum(jnp.log(jnp.abs(p)), LCLAMP) + RADIX * negp

  return pl.pallas_call(
      body,
      grid=(grid,),
      in_specs=[
          pl.BlockSpec((blk, 128), lambda i: (i, 0)),
          pl.BlockSpec((128, 128), lambda i: (0, 0)),
          pl.BlockSpec((128, 128), lambda i: (0, 0)),
          pl.BlockSpec((1, 128), lambda i: (0, 0)),
          pl.BlockSpec((blk, 128), lambda i: (i, 0)),
      ],
      out_specs=[
          pl.BlockSpec((blk, 128), lambda i: (i, 0)),
          pl.BlockSpec((blk, 128), lambda i: (i, 0)),
      ],
      out_shape=[
          jax.ShapeDtypeStruct((NPAD, 128), jnp.float32),
          jax.ShapeDtypeStruct((NPAD, 128), jnp.float32),
      ],
  )(featp, w1, w2a, b2, outdeg)


def _tc_combo(bond_tables, ewT):
  def body(bt_ref, e_ref, ew_ref, l_ref, c_ref):
    bt = bt_ref[...]
    t01 = (bt[0][:, None, :] + bt[1][None, :, :]).reshape(64, 128)
    t012 = (t01[:, None, :] + bt[2][None, :, :]).reshape(512, 128)
    ew = (t012[:, None, :] + bt[3][None, :, :]).reshape(4096, 128)
    ew_ref[...] = ew
    neg = jnp.where(ew < 0, 1.0, 0.0).astype(jnp.float32)
    l_ref[...] = jnp.maximum(jnp.log(jnp.abs(ew)), LCLAMP) + RADIX * neg
    ev = e_ref[...]
    c_ref[...] = ev[0:1] * 512 + ev[1:2] * 64 + ev[2:3] * 8 + ev[3:4]

  return pl.pallas_call(
      body,
      out_shape=[
          jax.ShapeDtypeStruct((4096, 128), jnp.float32),
          jax.ShapeDtypeStruct((4096, 128), jnp.float32),
          jax.ShapeDtypeStruct((1, EPAD), jnp.int32),
      ],
  )(bond_tables, ewT)


def _tc_post(hs, sv, v, indeg, n):
  blk = 512
  grid = NPAD // blk

  def body(hs_ref, s_ref, v_ref, id_ref, out_ref):
    sval = s_ref[...]
    cnt = jnp.floor(sval * (1.0 / RADIX) + 0.5)
    lg = sval - RADIX * cnt
    par = cnt - 2.0 * jnp.floor(cnt * 0.5)
    hp = (1.0 - 2.0 * par) * jnp.exp(lg)
    r = hs_ref[...] + jnp.dot(hp, v_ref[...], preferred_element_type=jnp.float32)
    nd = lax.rsqrt(jnp.clip(id_ref[...][:, 0:1], 1.0, None))
    out_ref[...] = r * nd

  return pl.pallas_call(
      body,
      grid=(grid,),
      in_specs=[
          pl.BlockSpec((blk, 128), lambda i: (i, 0)),
          pl.BlockSpec((blk, 128), lambda i: (i, 0)),
          pl.BlockSpec((128, 128), lambda i: (0, 0)),
          pl.BlockSpec((blk, 128), lambda i: (i, 0)),
      ],
      out_specs=pl.BlockSpec((blk, 128), lambda i: (i, 0)),
      out_shape=jax.ShapeDtypeStruct((n, 128), jnp.float32),
  )(hs, sv, v, indeg)


def _sc_main(src2, code2, dst2, nt_a, nt_b, ct_a, ct_b):
  @functools.partial(
      pl.kernel,
      out_type=[jax.ShapeDtypeStruct((NPAD, 128), jnp.float32),
                jax.ShapeDtypeStruct((NPAD, 128), jnp.float32)],
      mesh=_MESH,
      scratch_types=[
          pltpu.VMEM_SHARED((NPAD, 128), jnp.float32),
          pltpu.VMEM((STG, CW), jnp.int32),
          pltpu.VMEM((STG, CW), jnp.int32),
          pltpu.VMEM((STG, CW), jnp.int32),
          pltpu.VMEM((CW, 128), jnp.float32),
          pltpu.VMEM((CW, 128), jnp.float32),
          pltpu.VMEM((CW, 128), jnp.float32),
          pltpu.VMEM((CW, 128), jnp.float32),
          pltpu.SemaphoreType.DMA,
          pltpu.SemaphoreType.DMA,
          pltpu.SemaphoreType.DMA,
          pltpu.SemaphoreType.DMA,
          pltpu.SemaphoreType.DMA,
          pltpu.SemaphoreType.DMA,
      ],
  )
  def main_kernel(src_hbm, code_hbm, dst_hbm, nta_hbm, ntb_hbm, cta_hbm,
                  ctb_hbm, out0_hbm, out1_hbm, acc, sidx, cidx, didx, nbuf0, nbuf1,
                  cbuf0, cbuf1, gn0, gn1, gc0, gc1, ss0, ss1):
    c = lax.axis_index("c")
    s = lax.axis_index("s")
    nb = [nbuf0, nbuf1]
    cb = [cbuf0, cbuf1]
    gn = [gn0, gn1]
    gc = [gc0, gc1]
    ssem = [ss0, ss1]

    def zfill(i, _):
      for f in range(8):
        nbuf0[i, pl.ds(f * 16, 16)] = jnp.zeros((16,), jnp.float32)
      return 0
    lax.fori_loop(0, CW, zfill, 0)

    def zacc(t, _):
      pltpu.sync_copy(nbuf0, acc.at[pl.ds(s * RN + t * CW, CW)])
      return 0
    lax.fori_loop(0, RN // CW, zacc, 0)
    plsc.subcore_barrier()

    def run(node_tbl, combo_tbl, is_mul):
      def stage(t, _):
        base = s * RPC + t * STG
        pltpu.sync_copy(src_hbm.at[pl.ds(base, STG)], sidx)
        pltpu.sync_copy(code_hbm.at[pl.ds(base, STG)], cidx)
        pltpu.sync_copy(dst_hbm.at[pl.ds(base, STG)], didx)

        gath = {}
        scat = [None, None]
        gath[0] = (pltpu.async_copy(node_tbl.at[sidx.at[0]], nb[0], gn[0]),
                   pltpu.async_copy(combo_tbl.at[cidx.at[0]], cb[0], gc[0]))
        for j in range(STG):
          pj = j % 2
          if j + 1 < STG:
            nx = (j + 1) % 2
            if scat[nx] is not None:
              scat[nx].wait()
              scat[nx] = None
            gath[j + 1] = (
                pltpu.async_copy(node_tbl.at[sidx.at[j + 1]], nb[nx], gn[nx]),
                pltpu.async_copy(combo_tbl.at[cidx.at[j + 1]], cb[nx], gc[nx]))
          gath[j][0].wait()
          gath[j][1].wait()
          nbj = nb[pj]
          cbj = cb[pj]

          def edge(e, _, nbj=nbj, cbj=cbj):
            for f in range(8):
              d = pl.ds(f * 16, 16)
              if is_mul:
                nbj[e, d] = nbj[e, d] * cbj[e, d]
              else:
                nbj[e, d] = nbj[e, d] + cbj[e, d]
            return 0
          lax.fori_loop(0, CW, edge, 0)
          scat[pj] = pltpu.async_copy(nbj, acc.at[didx.at[j]], ssem[pj],
                                      add=True)
        for d_ in scat:
          if d_ is not None:
            d_.wait()
        return 0
      lax.fori_loop(0, RPC // STG, stage, 0)

    @pl.when(c == 0)
    def _():
      run(nta_hbm, cta_hbm, True)

    @pl.when(c == 1)
    def _():
      run(ntb_hbm, ctb_hbm, False)

    plsc.subcore_barrier()

    @pl.when(c == 0)
    def _():
      pltpu.sync_copy(acc.at[pl.ds(s * RN, RN)], out0_hbm.at[pl.ds(s * RN, RN)])

    @pl.when(c == 1)
    def _():
      pltpu.sync_copy(acc.at[pl.ds(s * RN, RN)], out1_hbm.at[pl.ds(s * RN, RN)])

  return main_kernel(src2, code2, dst2, nt_a, nt_b, ct_a, ct_b)


def kernel(feat, edge_index, edge_weight, w1, w2, v, bond_tables):
  n = feat.shape[0]
  e = edge_index.shape[1]

  src = edge_index[0]
  dst = edge_index[1]
  pad_idx = jnp.full((EPAD - e,), NPAD - 1, jnp.int32)
  src2 = jnp.concatenate([src, pad_idx]).reshape(ER, CW)
  dst2 = jnp.concatenate([dst, pad_idx]).reshape(ER, CW)
  ewT = jnp.concatenate(
      [edge_weight, jnp.zeros((EPAD - e, 4), jnp.int32)]).T

  outdeg, indeg = _sc_degrees(src2, dst2)

  a1, nt_b = _tc_node(feat, w1, w2[:128], w2[128:129], outdeg)
  ew, ct_b, codes = _tc_combo(bond_tables, ewT)
  codes = codes.reshape(ER, CW)

  hs, sv = _sc_main(src2, codes, dst2, a1, nt_b, ew, ct_b)

  return _tc_post(hs, sv, v, indeg, n)


# speedup vs baseline: 1.0512x; 1.0512x over previous
import functools

import jax
import jax.numpy as jnp
from jax import lax
from jax.experimental import pallas as pl
from jax.experimental.pallas import tpu as pltpu
from jax.experimental.pallas import tpu_sc as plsc

NPAD = 10240
EPAD = 163840
CW = 64
ER = EPAD // CW
NC, NS = 2, 16
RPC = ER // NS
STG = 40
RN = NPAD // NS
RADIX = 16384.0
LCLAMP = -30.0

_MESH = plsc.VectorSubcoreMesh(
    core_axis_name="c", subcore_axis_name="s", num_cores=NC, num_subcores=NS)


def _sc_degrees(src2, dst2):
  @functools.partial(
      pl.kernel,
      out_type=[jax.ShapeDtypeStruct((NPAD, 128), jnp.float32),
                jax.ShapeDtypeStruct((NPAD, 128), jnp.float32)],
      mesh=_MESH,
      scratch_types=[
          pltpu.VMEM_SHARED((NPAD, 128), jnp.float32),
          pltpu.VMEM((RPC, CW), jnp.int32),
          pltpu.VMEM((CW, 128), jnp.float32),
          pltpu.VMEM((CW, 128), jnp.float32),
          pltpu.SemaphoreType.DMA,
          pltpu.SemaphoreType.DMA,
      ],
  )
  def deg_kernel(src_hbm, dst_hbm, out0_hbm, out1_hbm, acc, idx, ones, zbuf,
                 sa, sb):
    c = lax.axis_index("c")
    s = lax.axis_index("s")

    def fill(i, _):
      for f in range(8):
        d = pl.ds(f * 16, 16)
        ones[i, d] = jnp.full((16,), 1.0, jnp.float32)
        zbuf[i, d] = jnp.zeros((16,), jnp.float32)
      return 0
    lax.fori_loop(0, CW, fill, 0)

    def zacc(t, _):
      pltpu.sync_copy(zbuf, acc.at[pl.ds(s * RN + t * CW, CW)])
      return 0
    lax.fori_loop(0, RN // CW, zacc, 0)
    plsc.subcore_barrier()

    def scan(idx_hbm):
      pltpu.sync_copy(idx_hbm.at[pl.ds(s * RPC, RPC)], idx)
      sems = [sa, sb]
      descs = [None, None]
      for j in range(RPC):
        pj = j % 2
        if descs[pj] is not None:
          descs[pj].wait()
        descs[pj] = pltpu.async_copy(ones, acc.at[idx.at[j]], sems[pj],
                                     add=True)
      for d_ in descs:
        if d_ is not None:
          d_.wait()

    @pl.when(c == 0)
    def _():
      scan(src_hbm)

    @pl.when(c == 1)
    def _():
      scan(dst_hbm)

    plsc.subcore_barrier()

    @pl.when(c == 0)
    def _():
      pltpu.sync_copy(acc.at[pl.ds(s * RN, RN)], out0_hbm.at[pl.ds(s * RN, RN)])

    @pl.when(c == 1)
    def _():
      pltpu.sync_copy(acc.at[pl.ds(s * RN, RN)], out1_hbm.at[pl.ds(s * RN, RN)])

  return deg_kernel(src2, dst2)


def _tc_node(featp, w1, w2a, b2, outdeg):
  blk = 512
  grid = NPAD // blk

  def body(x_ref, w1_ref, w2a_ref, b2_ref, od_ref, a1_ref, l_ref):
    x = x_ref[...]
    s = lax.rsqrt(jnp.clip(od_ref[...][:, 0:1], 1.0, None))
    a1 = jnp.dot(x, w1_ref[...], preferred_element_type=jnp.float32) * s
    z = jnp.dot(x, w2a_ref[...], preferred_element_type=jnp.float32) * s
    p = jnp.tanh(z + b2_ref[...])
    a1_ref[...] = a1
    negp = jnp.where(p < 0, 1.0, 0.0).astype(jnp.float32)
    l_ref[...] = jnp.maximum(jnp.log(jnp.abs(p)), LCLAMP) + RADIX * negp

  return pl.pallas_call(
      body,
      grid=(grid,),
      in_specs=[
          pl.BlockSpec((blk, 128), lambda i: (i, 0)),
          pl.BlockSpec((128, 128), lambda i: (0, 0)),
          pl.BlockSpec((128, 128), lambda i: (0, 0)),
          pl.BlockSpec((1, 128), lambda i: (0, 0)),
          pl.BlockSpec((blk, 128), lambda i: (i, 0)),
      ],
      out_specs=[
          pl.BlockSpec((blk, 128), lambda i: (i, 0)),
          pl.BlockSpec((blk, 128), lambda i: (i, 0)),
      ],
      out_shape=[
          jax.ShapeDtypeStruct((NPAD, 128), jnp.float32),
          jax.ShapeDtypeStruct((NPAD, 128), jnp.float32),
      ],
  )(featp, w1, w2a, b2, outdeg)


def _tc_combo(bond_tables, ewT):
  def body(bt_ref, e_ref, ew_ref, l_ref, c_ref):
    bt = bt_ref[...]
    t01 = (bt[0][:, None, :] + bt[1][None, :, :]).reshape(64, 128)
    t012 = (t01[:, None, :] + bt[2][None, :, :]).reshape(512, 128)
    ew = (t012[:, None, :] + bt[3][None, :, :]).reshape(4096, 128)
    ew_ref[...] = ew
    neg = jnp.where(ew < 0, 1.0, 0.0).astype(jnp.float32)
    l_ref[...] = jnp.maximum(jnp.log(jnp.abs(ew)), LCLAMP) + RADIX * neg
    ev = e_ref[...]
    c_ref[...] = ev[0:1] * 512 + ev[1:2] * 64 + ev[2:3] * 8 + ev[3:4]

  return pl.pallas_call(
      body,
      out_shape=[
          jax.ShapeDtypeStruct((4096, 128), jnp.float32),
          jax.ShapeDtypeStruct((4096, 128), jnp.float32),
          jax.ShapeDtypeStruct((1, EPAD), jnp.int32),
      ],
  )(bond_tables, ewT)


def _tc_post(hs, sv, v, indeg, n):
  blk = 512
  grid = NPAD // blk

  def body(hs_ref, s_ref, v_ref, id_ref, out_ref):
    sval = s_ref[...]
    cnt = jnp.floor(sval * (1.0 / RADIX) + 0.5)
    lg = sval - RADIX * cnt
    par = cnt - 2.0 * jnp.floor(cnt * 0.5)
    hp = (1.0 - 2.0 * par) * jnp.exp(lg)
    r = hs_ref[...] + jnp.dot(hp, v_ref[...], preferred_element_type=jnp.float32)
    nd = lax.rsqrt(jnp.clip(id_ref[...][:, 0:1], 1.0, None))
    out_ref[...] = r * nd

  return pl.pallas_call(
      body,
      grid=(grid,),
      in_specs=[
          pl.BlockSpec((blk, 128), lambda i: (i, 0)),
          pl.BlockSpec((blk, 128), lambda i: (i, 0)),
          pl.BlockSpec((128, 128), lambda i: (0, 0)),
          pl.BlockSpec((blk, 128), lambda i: (i, 0)),
      ],
      out_specs=pl.BlockSpec((blk, 128), lambda i: (i, 0)),
      out_shape=jax.ShapeDtypeStruct((n, 128), jnp.float32),
  )(hs, sv, v, indeg)


def _sc_main(src2, code2, dst2, nt_a, nt_b, ct_a, ct_b):
  @functools.partial(
      pl.kernel,
      out_type=[jax.ShapeDtypeStruct((NPAD, 128), jnp.float32),
                jax.ShapeDtypeStruct((NPAD, 128), jnp.float32)],
      mesh=_MESH,
      scratch_types=[
          pltpu.VMEM_SHARED((NPAD, 128), jnp.float32),
          pltpu.VMEM((STG, CW), jnp.int32),
          pltpu.VMEM((STG, CW), jnp.int32),
          pltpu.VMEM((STG, CW), jnp.int32),
          pltpu.VMEM((CW, 128), jnp.float32),
          pltpu.VMEM((CW, 128), jnp.float32),
          pltpu.VMEM((CW, 128), jnp.float32),
          pltpu.VMEM((CW, 128), jnp.float32),
          pltpu.SemaphoreType.DMA,
          pltpu.SemaphoreType.DMA,
          pltpu.SemaphoreType.DMA,
          pltpu.SemaphoreType.DMA,
          pltpu.SemaphoreType.DMA,
          pltpu.SemaphoreType.DMA,
      ],
  )
  def main_kernel(src_hbm, code_hbm, dst_hbm, nta_hbm, ntb_hbm, cta_hbm,
                  ctb_hbm, out0_hbm, out1_hbm, acc, sidx, cidx, didx, nbuf0, nbuf1,
                  cbuf0, cbuf1, gn0, gn1, gc0, gc1, ss0, ss1):
    c = lax.axis_index("c")
    s = lax.axis_index("s")
    nb = [nbuf0, nbuf1]
    cb = [cbuf0, cbuf1]
    gn = [gn0, gn1]
    gc = [gc0, gc1]
    ssem = [ss0, ss1]

    def zfill(i, _):
      for f in range(8):
        nbuf0[i, pl.ds(f * 16, 16)] = jnp.zeros((16,), jnp.float32)
      return 0
    lax.fori_loop(0, CW, zfill, 0)

    def zacc(t, _):
      pltpu.sync_copy(nbuf0, acc.at[pl.ds(s * RN + t * CW, CW)])
      return 0
    lax.fori_loop(0, RN // CW, zacc, 0)
    plsc.subcore_barrier()

    def run(node_tbl, combo_tbl, is_mul):
      def stage(t, _):
        base = s * RPC + t * STG
        pltpu.sync_copy(src_hbm.at[pl.ds(base, STG)], sidx)
        pltpu.sync_copy(code_hbm.at[pl.ds(base, STG)], cidx)
        pltpu.sync_copy(dst_hbm.at[pl.ds(base, STG)], didx)

        gath = {}
        scat = [None, None]
        gath[0] = (pltpu.async_copy(node_tbl.at[sidx.at[0]], nb[0], gn[0]),
                   pltpu.async_copy(combo_tbl.at[cidx.at[0]], cb[0], gc[0]))
        for j in range(STG):
          pj = j % 2
          if j + 1 < STG:
            nx = (j + 1) % 2
            if scat[nx] is not None:
              scat[nx].wait()
              scat[nx] = None
            gath[j + 1] = (
                pltpu.async_copy(node_tbl.at[sidx.at[j + 1]], nb[nx], gn[nx]),
                pltpu.async_copy(combo_tbl.at[cidx.at[j + 1]], cb[nx], gc[nx]))
          gath[j][0].wait()
          gath[j][1].wait()
          nbj = nb[pj]
          cbj = cb[pj]

          def edge(e, _, nbj=nbj, cbj=cbj):
            for f in range(8):
              d = pl.ds(f * 16, 16)
              if is_mul:
                nbj[e, d] = nbj[e, d] * cbj[e, d]
              else:
                nbj[e, d] = nbj[e, d] + cbj[e, d]
            return 0
          lax.fori_loop(0, CW, edge, 0)
          scat[pj] = pltpu.async_copy(nbj, acc.at[didx.at[j]], ssem[pj],
                                      add=True)
        for d_ in scat:
          if d_ is not None:
            d_.wait()
        return 0
      lax.fori_loop(0, RPC // STG, stage, 0)

    @pl.when(c == 0)
    def _():
      run(nta_hbm, cta_hbm, True)

    @pl.when(c == 1)
    def _():
      run(ntb_hbm, ctb_hbm, False)

    plsc.subcore_barrier()

    @pl.when(c == 0)
    def _():
      pltpu.sync_copy(acc.at[pl.ds(s * RN, RN)], out0_hbm.at[pl.ds(s * RN, RN)])

    @pl.when(c == 1)
    def _():
      pltpu.sync_copy(acc.at[pl.ds(s * RN, RN)], out1_hbm.at[pl.ds(s * RN, RN)])

  return main_kernel(src2, code2, dst2, nt_a, nt_b, ct_a, ct_b)


def kernel(feat, edge_index, edge_weight, w1, w2, v, bond_tables):
  n = feat.shape[0]
  e = edge_index.shape[1]

  src = edge_index[0]
  dst = edge_index[1]
  pad_idx = jnp.full((EPAD - e,), NPAD - 1, jnp.int32)
  src2 = jnp.concatenate([src, pad_idx]).reshape(ER, CW)
  dst2 = jnp.concatenate([dst, pad_idx]).reshape(ER, CW)
  ewT = jnp.concatenate(
      [edge_weight, jnp.zeros((EPAD - e, 4), jnp.int32)]).T

  featp = jnp.pad(feat, ((0, NPAD - n), (0, 0)))

  outdeg, indeg = _sc_degrees(src2, dst2)

  a1, nt_b = _tc_node(featp, w1, w2[:128], w2[128:129], outdeg)
  ew, ct_b, codes = _tc_combo(bond_tables, ewT)
  codes = codes.reshape(ER, CW)

  hs, sv = _sc_main(src2, codes, dst2, a1, nt_b, ew, ct_b)

  return _tc_post(hs, sv, v, indeg, n)
